# Initial kernel scaffold; baseline (speedup 1.0000x reference)
import jax
import jax.numpy as jnp
from jax.experimental import pallas as pl


def kernel(x, edge_index, edge_attr, batch, W_atom, b_atom, W_e1, b_e1, W_e2, b_e2, W_root, b_root, W_fp, b_fp):
    raise NotImplementedError("write your pallas kernel here")


# speedup vs baseline: 3.5111x; 3.5111x over previous
import functools

import jax
import jax.numpy as jnp
from jax import lax
from jax.experimental import pallas as pl
from jax.experimental.pallas import tpu as pltpu
from jax.experimental.pallas import tpu_sc as plsc

N_NODES = 10000
N_EDGES = 160000
NODE_DIM = 128
BOND_DIM = 16
CONV_DIM = 16
INTERNAL_DIM = 128
FP_DIM = 2048
NUM_GRAPHS = 64

NC = 2
NS = 16
NW = NC * NS
CHUNK = 128
PER_TILE = 5120
E_PAD = NW * PER_TILE
NCHUNK = PER_TILE // CHUNK
GROUP = 8
NGROUP = NCHUNK // GROUP
ROWS_PER_TILE = N_NODES // NS

EB = 2048
NE_BLOCKS = E_PAD // EB
NB = 1000
NN_BLOCKS = N_NODES // NB

_f32 = jnp.float32

_sc_mesh = plsc.VectorSubcoreMesh(core_axis_name="c", subcore_axis_name="s")


@functools.partial(
    pl.kernel,
    mesh=_sc_mesh,
    out_type=jax.ShapeDtypeStruct((E_PAD, CONV_DIM), _f32),
    scratch_types=[
        pltpu.VMEM((NCHUNK, CHUNK), jnp.int32),
        pltpu.VMEM((GROUP * CHUNK, CONV_DIM), _f32),
        pltpu.SemaphoreType.DMA,
    ],
    compiler_params=pltpu.CompilerParams(use_tc_tiling_on_sc=False),
)
def _sc_gather(table_hbm, idx_hbm, out_hbm, idx_v, rows_v, sem):
    wid = lax.axis_index("c") * NS + lax.axis_index("s")
    pltpu.sync_copy(idx_hbm.at[pl.ds(wid * NCHUNK, NCHUNK)], idx_v)

    def grp(g, carry):
        copies = []
        for j in range(GROUP):
            cp = pltpu.async_copy(
                table_hbm.at[idx_v.at[g * GROUP + j]],
                rows_v.at[pl.ds(j * CHUNK, CHUNK)],
                sem,
            )
            copies.append(cp)
        for cp in copies:
            cp.wait()
        pltpu.sync_copy(
            rows_v,
            out_hbm.at[pl.ds(wid * PER_TILE + g * (GROUP * CHUNK), GROUP * CHUNK)],
        )
        return carry

    lax.fori_loop(0, NGROUP, grp, 0)


@functools.partial(
    pl.kernel,
    mesh=_sc_mesh,
    out_type=jax.ShapeDtypeStruct((NC, N_NODES, CONV_DIM), _f32),
    scratch_types=[
        pltpu.VMEM((NCHUNK, CHUNK), jnp.int32),
        pltpu.VMEM((CHUNK, CONV_DIM), _f32),
        pltpu.VMEM((ROWS_PER_TILE, CONV_DIM), _f32),
        pltpu.VMEM_SHARED((N_NODES, CONV_DIM), _f32),
        pltpu.SemaphoreType.DMA,
    ],
    compiler_params=pltpu.CompilerParams(use_tc_tiling_on_sc=False),
)
def _sc_scatter(m_hbm, idx_hbm, zeros_hbm, out_hbm, idx_v, m_v, bounce_v, acc_sh, sem):
    cid = lax.axis_index("c")
    sid = lax.axis_index("s")
    wid = cid * NS + sid
    rows = pl.ds(sid * ROWS_PER_TILE, ROWS_PER_TILE)
    pltpu.sync_copy(zeros_hbm.at[rows], bounce_v)
    pltpu.sync_copy(bounce_v, acc_sh.at[rows])
    plsc.subcore_barrier()

    pltpu.sync_copy(idx_hbm.at[pl.ds(wid * NCHUNK, NCHUNK)], idx_v)

    def chunk(j, carry):
        pltpu.sync_copy(m_hbm.at[pl.ds(wid * PER_TILE + j * CHUNK, CHUNK)], m_v)
        pltpu.sync_copy(m_v, acc_sh.at[idx_v.at[j]], add=True)
        return carry

    lax.fori_loop(0, NCHUNK, chunk, 0)
    plsc.subcore_barrier()

    pltpu.sync_copy(acc_sh.at[rows], bounce_v)
    pltpu.sync_copy(bounce_v, out_hbm.at[cid, rows])


def _msg0_body(x_ref, wa_ref, ba_ref, o_ref):
    o_ref[...] = jnp.maximum(
        jnp.dot(x_ref[...], wa_ref[...], preferred_element_type=_f32) + ba_ref[...],
        0.0,
    )


def _we_m0_body(ea_ref, g_ref, we1_ref, be1_ref, we2_ref, be2_ref, r_ref, s_ref,
                we_ref, m_ref):
    ea = ea_ref[...]
    h = jnp.maximum(
        jnp.dot(ea, we1_ref[...], preferred_element_type=_f32) + be1_ref[...], 0.0)
    we = jnp.dot(h, we2_ref[...], preferred_element_type=_f32) + be2_ref[...]
    we_ref[...] = we
    mask = (ea[:, BOND_DIM - 1:BOND_DIM] == 0.0).astype(_f32)
    gm = g_ref[...] * mask
    p = jnp.dot(gm, r_ref[...], preferred_element_type=_f32) * we
    m_ref[...] = jnp.dot(p, s_ref[...], preferred_element_type=_f32)


def _m1_body(ea_ref, g_ref, we_ref, r_ref, s_ref, m_ref):
    mask = (ea_ref[:, BOND_DIM - 1:BOND_DIM] <= 1.0).astype(_f32)
    gm = g_ref[...] * mask
    p = jnp.dot(gm, r_ref[...], preferred_element_type=_f32) * we_ref[...]
    m_ref[...] = jnp.dot(p, s_ref[...], preferred_element_type=_f32)


def _update_body(msg_ref, wr_ref, br_ref, p_ref, o_ref, *, scale):
    agg = p_ref[0] + p_ref[1]
    lin = jnp.dot(msg_ref[...], wr_ref[...], preferred_element_type=_f32) + br_ref[...]
    o_ref[...] = jnp.maximum(scale * lin + agg, 0.0)


def _fp_body(msg_ref, b_ref, wfp_ref, bfp_ref, o_ref):
    i = pl.program_id(0)
    y = jnp.maximum(
        jnp.dot(msg_ref[...], wfp_ref[...], preferred_element_type=_f32)
        + bfp_ref[...], 0.0)
    gids = lax.broadcasted_iota(jnp.int32, (NUM_GRAPHS, NB), 0)
    oh = (b_ref[0] == gids).astype(_f32)
    contrib = jnp.dot(oh, y, preferred_element_type=_f32)

    @pl.when(i == 0)
    def _():
        o_ref[...] = contrib

    @pl.when(i > 0)
    def _():
        o_ref[...] += contrib


def kernel(x, edge_index, edge_attr, batch, W_atom, b_atom, W_e1, b_e1, W_e2, b_e2,
           W_root, b_root, W_fp, b_fp):
    pad = E_PAD - N_EDGES
    src2 = jnp.concatenate(
        [edge_index[0], jnp.zeros((pad,), jnp.int32)]).reshape(E_PAD // CHUNK, CHUNK)
    dst2 = jnp.concatenate(
        [edge_index[1], jnp.zeros((pad,), jnp.int32)]).reshape(E_PAD // CHUNK, CHUNK)
    ea_p = jnp.concatenate(
        [edge_attr, jnp.full((pad, BOND_DIM), 2.0, _f32)])
    zeros_nt = jnp.zeros((N_NODES, CONV_DIM), _f32)
    batch3 = batch.reshape(NN_BLOCKS, 1, NB)

    ba = b_atom.reshape(1, CONV_DIM)
    be1 = b_e1.reshape(1, INTERNAL_DIM)
    be2 = b_e2.reshape(1, CONV_DIM * CONV_DIM)
    br = b_root.reshape(1, CONV_DIM)
    bfp = b_fp.reshape(1, FP_DIM)

    cols = jnp.arange(CONV_DIM * CONV_DIM)
    r_sel = (cols[None, :] // CONV_DIM == jnp.arange(CONV_DIM)[:, None]).astype(_f32)
    s_sel = (cols[:, None] % CONV_DIM == jnp.arange(CONV_DIM)[None, :]).astype(_f32)

    msg0 = pl.pallas_call(
        _msg0_body,
        out_shape=jax.ShapeDtypeStruct((N_NODES, CONV_DIM), _f32),
    )(x, W_atom, ba)

    full = lambda shape: pl.BlockSpec(shape, lambda i: (0,) * len(shape))
    eb16 = pl.BlockSpec((EB, CONV_DIM), lambda i: (i, 0))
    eb256 = pl.BlockSpec((EB, CONV_DIM * CONV_DIM), lambda i: (i, 0))

    g0 = _sc_gather(msg0, src2)
    we, m0 = pl.pallas_call(
        _we_m0_body,
        grid=(NE_BLOCKS,),
        in_specs=[eb16, eb16, full((BOND_DIM, INTERNAL_DIM)),
                  full((1, INTERNAL_DIM)), full((INTERNAL_DIM, CONV_DIM * CONV_DIM)),
                  full((1, CONV_DIM * CONV_DIM)),
                  full((CONV_DIM, CONV_DIM * CONV_DIM)),
                  full((CONV_DIM * CONV_DIM, CONV_DIM))],
        out_specs=[eb256, eb16],
        out_shape=[jax.ShapeDtypeStruct((E_PAD, CONV_DIM * CONV_DIM), _f32),
                   jax.ShapeDtypeStruct((E_PAD, CONV_DIM), _f32)],
    )(ea_p, g0, W_e1, be1, W_e2, be2, r_sel, s_sel)
    parts0 = _sc_scatter(m0, dst2, zeros_nt)
    msg1 = pl.pallas_call(
        functools.partial(_update_body, scale=1.0),
        out_shape=jax.ShapeDtypeStruct((N_NODES, CONV_DIM), _f32),
    )(msg0, W_root, br, parts0)

    g1 = _sc_gather(msg1, src2)
    m1 = pl.pallas_call(
        _m1_body,
        grid=(NE_BLOCKS,),
        in_specs=[eb16, eb16, eb256,
                  full((CONV_DIM, CONV_DIM * CONV_DIM)),
                  full((CONV_DIM * CONV_DIM, CONV_DIM))],
        out_specs=eb16,
        out_shape=jax.ShapeDtypeStruct((E_PAD, CONV_DIM), _f32),
    )(ea_p, g1, we, r_sel, s_sel)
    parts1 = _sc_scatter(m1, dst2, zeros_nt)
    msg2 = pl.pallas_call(
        functools.partial(_update_body, scale=2.0),
        out_shape=jax.ShapeDtypeStruct((N_NODES, CONV_DIM), _f32),
    )(msg1, W_root, br, parts1)

    fingerprint = pl.pallas_call(
        _fp_body,
        grid=(NN_BLOCKS,),
        in_specs=[pl.BlockSpec((NB, CONV_DIM), lambda i: (i, 0)),
                  pl.BlockSpec((1, 1, NB), lambda i: (i, 0, 0)),
                  full((CONV_DIM, FP_DIM)), full((1, FP_DIM))],
        out_specs=pl.BlockSpec((NUM_GRAPHS, FP_DIM), lambda i: (0, 0)),
        out_shape=jax.ShapeDtypeStruct((NUM_GRAPHS, FP_DIM), _f32),
    )(msg2, batch3, W_fp, bfp)
    return fingerprint


# speedup vs baseline: 4.0123x; 1.1428x over previous
import functools

import jax
import jax.numpy as jnp
from jax import lax
from jax.experimental import pallas as pl
from jax.experimental.pallas import tpu as pltpu
from jax.experimental.pallas import tpu_sc as plsc

N_NODES = 10000
N_EDGES = 160000
NODE_DIM = 128
BOND_DIM = 16
CONV_DIM = 16
INTERNAL_DIM = 128
FP_DIM = 2048
NUM_GRAPHS = 64

NC = 2
NS = 16
NW = NC * NS
CHUNK = 128
PER_TILE = N_EDGES // NW
NFULL = PER_TILE // CHUNK
TAIL = PER_TILE - NFULL * CHUNK
GROUP = 8
ROWS_PER_TILE = N_NODES // NS

EB = 2000
NE_BLOCKS = N_EDGES // EB
NB = 1000
NN_BLOCKS = N_NODES // NB

_f32 = jnp.float32

_sc_mesh = plsc.VectorSubcoreMesh(core_axis_name="c", subcore_axis_name="s")
_sc_params = pltpu.CompilerParams(use_tc_tiling_on_sc=False)


@functools.partial(
    pl.kernel,
    mesh=_sc_mesh,
    out_type=jax.ShapeDtypeStruct((N_EDGES, CONV_DIM), _f32),
    scratch_types=[
        pltpu.VMEM((PER_TILE,), jnp.int32),
        pltpu.VMEM((GROUP * CHUNK, CONV_DIM), _f32),
        pltpu.SemaphoreType.DMA,
    ],
    compiler_params=_sc_params,
)
def _sc_gather(table_hbm, idx_hbm, out_hbm, idx_v, rows_v, sem):
    wid = lax.axis_index("c") * NS + lax.axis_index("s")
    base = wid * PER_TILE
    pltpu.sync_copy(idx_hbm.at[pl.ds(base, PER_TILE)], idx_v)

    def grp(g, carry):
        copies = []
        for j in range(GROUP):
            cp = pltpu.async_copy(
                table_hbm.at[idx_v.at[pl.ds((g * GROUP + j) * CHUNK, CHUNK)]],
                rows_v.at[pl.ds(j * CHUNK, CHUNK)],
                sem,
            )
            copies.append(cp)
        for cp in copies:
            cp.wait()
        pltpu.sync_copy(
            rows_v.at[pl.ds(0, GROUP * CHUNK)],
            out_hbm.at[pl.ds(base + g * (GROUP * CHUNK), GROUP * CHUNK)],
        )
        return carry

    lax.fori_loop(0, NFULL // GROUP, grp, 0)
    rest = NFULL - (NFULL // GROUP) * GROUP
    done = (NFULL // GROUP) * GROUP * CHUNK
    copies = []
    for j in range(rest):
        copies.append(pltpu.async_copy(
            table_hbm.at[idx_v.at[pl.ds(done + j * CHUNK, CHUNK)]],
            rows_v.at[pl.ds(j * CHUNK, CHUNK)], sem))
    copies.append(pltpu.async_copy(
        table_hbm.at[idx_v.at[pl.ds(done + rest * CHUNK, TAIL)]],
        rows_v.at[pl.ds(rest * CHUNK, TAIL)], sem))
    for cp in copies:
        cp.wait()
    pltpu.sync_copy(
        rows_v.at[pl.ds(0, rest * CHUNK + TAIL)],
        out_hbm.at[pl.ds(base + done, rest * CHUNK + TAIL)],
    )


@functools.partial(
    pl.kernel,
    mesh=_sc_mesh,
    out_type=jax.ShapeDtypeStruct((NC, N_NODES, CONV_DIM), _f32),
    scratch_types=[
        pltpu.VMEM((NFULL, CHUNK), jnp.int32),
        pltpu.VMEM((TAIL,), jnp.int32),
        pltpu.VMEM((CHUNK, CONV_DIM), _f32),
        pltpu.VMEM((ROWS_PER_TILE, CONV_DIM), _f32),
        pltpu.VMEM_SHARED((N_NODES, CONV_DIM), _f32),
        pltpu.SemaphoreType.DMA,
    ],
    compiler_params=_sc_params,
)
def _sc_scatter(m_hbm, idx2_hbm, idx_hbm, zeros_hbm, out_hbm,
                idx_v, idxt_v, m_v, bounce_v, acc_sh, sem):
    cid = lax.axis_index("c")
    sid = lax.axis_index("s")
    wid = cid * NS + sid
    base = wid * PER_TILE
    rows = pl.ds(sid * ROWS_PER_TILE, ROWS_PER_TILE)
    pltpu.sync_copy(zeros_hbm.at[rows], bounce_v)
    pltpu.sync_copy(bounce_v, acc_sh.at[rows])
    plsc.subcore_barrier()

    pltpu.sync_copy(idx2_hbm.at[pl.ds(wid * NFULL, NFULL)], idx_v)
    pltpu.sync_copy(idx_hbm.at[pl.ds(base + NFULL * CHUNK, TAIL)], idxt_v)

    def chunk(j, carry):
        pltpu.sync_copy(m_hbm.at[pl.ds(base + j * CHUNK, CHUNK)], m_v)
        pltpu.sync_copy(m_v, acc_sh.at[idx_v.at[j]], add=True)
        return carry

    lax.fori_loop(0, NFULL, chunk, 0)
    pltpu.sync_copy(m_hbm.at[pl.ds(base + NFULL * CHUNK, TAIL)],
                    m_v.at[pl.ds(0, TAIL)])
    pltpu.sync_copy(m_v.at[pl.ds(0, TAIL)], acc_sh.at[idxt_v], add=True)
    plsc.subcore_barrier()

    pltpu.sync_copy(acc_sh.at[rows], bounce_v)
    pltpu.sync_copy(bounce_v, out_hbm.at[cid, rows])


def _msg0_body(x_ref, wa_ref, ba_ref, o_ref):
    o_ref[...] = jnp.maximum(
        jnp.dot(x_ref[...], wa_ref[...], preferred_element_type=_f32) + ba_ref[...],
        0.0,
    )


def _edge_body(ea_ref, g_ref, we1_ref, be1_ref, we2_ref, be2_ref, r_ref, s_ref,
               m_ref, *, masked):
    ea = ea_ref[...]
    h = jnp.maximum(
        jnp.dot(ea, we1_ref[...], preferred_element_type=_f32) + be1_ref[...], 0.0)
    we = jnp.dot(h, we2_ref[...], preferred_element_type=_f32) + be2_ref[...]
    g = g_ref[...]
    if masked:
        g = g * (ea[:, BOND_DIM - 1:BOND_DIM] == 0.0).astype(_f32)
    p = jnp.dot(g, r_ref[...], preferred_element_type=_f32) * we
    m_ref[...] = jnp.dot(p, s_ref[...], preferred_element_type=_f32)


def _update_body(msg_ref, wr_ref, br_ref, p_ref, o_ref, *, scale):
    agg = p_ref[0] + p_ref[1]
    lin = jnp.dot(msg_ref[...], wr_ref[...], preferred_element_type=_f32) + br_ref[...]
    o_ref[...] = jnp.maximum(scale * lin + agg, 0.0)


def _fp_body(msg_ref, b_ref, wfp_ref, bfp_ref, o_ref):
    i = pl.program_id(0)
    y = jnp.maximum(
        jnp.dot(msg_ref[...], wfp_ref[...], preferred_element_type=_f32)
        + bfp_ref[...], 0.0)
    gids = lax.broadcasted_iota(jnp.int32, (NUM_GRAPHS, NB), 0)
    oh = (b_ref[0] == gids).astype(_f32)
    contrib = jnp.dot(oh, y, preferred_element_type=_f32)

    @pl.when(i == 0)
    def _():
        o_ref[...] = contrib

    @pl.when(i > 0)
    def _():
        o_ref[...] += contrib


def _edge_kernel(masked):
    full = lambda shape: pl.BlockSpec(shape, lambda i: (0,) * len(shape))
    eb16 = pl.BlockSpec((EB, CONV_DIM), lambda i: (i, 0))
    return pl.pallas_call(
        functools.partial(_edge_body, masked=masked),
        grid=(NE_BLOCKS,),
        in_specs=[eb16, eb16, full((BOND_DIM, INTERNAL_DIM)),
                  full((1, INTERNAL_DIM)),
                  full((INTERNAL_DIM, CONV_DIM * CONV_DIM)),
                  full((1, CONV_DIM * CONV_DIM)),
                  full((CONV_DIM, CONV_DIM * CONV_DIM)),
                  full((CONV_DIM * CONV_DIM, CONV_DIM))],
        out_specs=eb16,
        out_shape=jax.ShapeDtypeStruct((N_EDGES, CONV_DIM), _f32),
    )


def kernel(x, edge_index, edge_attr, batch, W_atom, b_atom, W_e1, b_e1, W_e2, b_e2,
           W_root, b_root, W_fp, b_fp):
    src = edge_index[0]
    dst = edge_index[1]
    dst2 = dst.reshape(NW, PER_TILE)[:, :NFULL * CHUNK].reshape(NW * NFULL, CHUNK)
    zeros_nt = jnp.zeros((N_NODES, CONV_DIM), _f32)
    batch3 = batch.reshape(NN_BLOCKS, 1, NB)

    ba = b_atom.reshape(1, CONV_DIM)
    be1 = b_e1.reshape(1, INTERNAL_DIM)
    be2 = b_e2.reshape(1, CONV_DIM * CONV_DIM)
    br = b_root.reshape(1, CONV_DIM)
    bfp = b_fp.reshape(1, FP_DIM)

    cols = jnp.arange(CONV_DIM * CONV_DIM)
    r_sel = (cols[None, :] // CONV_DIM == jnp.arange(CONV_DIM)[:, None]).astype(_f32)
    s_sel = (cols[:, None] % CONV_DIM == jnp.arange(CONV_DIM)[None, :]).astype(_f32)

    msg0 = pl.pallas_call(
        _msg0_body,
        out_shape=jax.ShapeDtypeStruct((N_NODES, CONV_DIM), _f32),
    )(x, W_atom, ba)

    g0 = _sc_gather(msg0, src)
    m0 = _edge_kernel(masked=True)(edge_attr, g0, W_e1, be1, W_e2, be2, r_sel, s_sel)
    parts0 = _sc_scatter(m0, dst2, dst, zeros_nt)
    msg1 = pl.pallas_call(
        functools.partial(_update_body, scale=1.0),
        out_shape=jax.ShapeDtypeStruct((N_NODES, CONV_DIM), _f32),
    )(msg0, W_root, br, parts0)

    g1 = _sc_gather(msg1, src)
    m1 = _edge_kernel(masked=False)(edge_attr, g1, W_e1, be1, W_e2, be2, r_sel, s_sel)
    parts1 = _sc_scatter(m1, dst2, dst, zeros_nt)
    msg2 = pl.pallas_call(
        functools.partial(_update_body, scale=2.0),
        out_shape=jax.ShapeDtypeStruct((N_NODES, CONV_DIM), _f32),
    )(msg1, W_root, br, parts1)

    full = lambda shape: pl.BlockSpec(shape, lambda i: (0,) * len(shape))
    fingerprint = pl.pallas_call(
        _fp_body,
        grid=(NN_BLOCKS,),
        in_specs=[pl.BlockSpec((NB, CONV_DIM), lambda i: (i, 0)),
                  pl.BlockSpec((1, 1, NB), lambda i: (i, 0, 0)),
                  full((CONV_DIM, FP_DIM)), full((1, FP_DIM))],
        out_specs=pl.BlockSpec((NUM_GRAPHS, FP_DIM), lambda i: (0, 0)),
        out_shape=jax.ShapeDtypeStruct((NUM_GRAPHS, FP_DIM), _f32),
    )(msg2, batch3, W_fp, bfp)
    return fingerprint
